# baseline (speedup 1.0000x reference)
import jax
import jax.numpy as jnp
from jax import lax
from jax.experimental import pallas as pl
from jax.experimental.pallas import tpu as pltpu
from jax.experimental.pallas import tpu_sc as plsc

N = 10000
D = 128
DH = 64
NP = 10240
NS = 16
RPT = NP // NS
CH = 128
E = 320000
KB = 2
NCH32 = 80
ETOT = 32 * NCH32 * CH
NCH = 2 * NCH32
NBLK = NCH // KB
PAD_IDX = NP - 1

_mesh = plsc.VectorSubcoreMesh(core_axis_name="c", subcore_axis_name="s")

_f32 = jnp.float32

_sc_params = pltpu.CompilerParams(use_tc_tiling_on_sc=False)


def _deg_body(dst_hbm, deg_out, idx_v, ones_v, zbuf, deg_sp):
    c = lax.axis_index("c")
    s = lax.axis_index("s")

    def _fill(i, _):
        ones_v[pl.ds(i * 16, 16)] = jnp.ones((16,), _f32)
        zbuf[pl.ds(i * 16, 16)] = jnp.zeros((16,), _f32)
        return ()

    lax.fori_loop(0, RPT // 16, _fill, ())

    pltpu.sync_copy(dst_hbm.at[c, s], idx_v)
    pltpu.sync_copy(zbuf, deg_sp.at[pl.ds(s * RPT, RPT)])
    plsc.subcore_barrier()

    def _chunk(j, _):
        pltpu.sync_copy(ones_v.at[pl.ds(0, CH)], deg_sp.at[idx_v.at[j]],
                        add=True)
        return ()

    lax.fori_loop(0, NCH32, _chunk, ())
    plsc.subcore_barrier()

    @pl.when(s == 0)
    def _():
        pltpu.sync_copy(deg_sp, deg_out.at[c])


_deg_call = pl.kernel(
    _deg_body,
    out_type=jax.ShapeDtypeStruct((2, NP), _f32),
    mesh=_mesh,
    scratch_types=[
        pltpu.VMEM((NCH32, CH), jnp.int32),
        pltpu.VMEM((RPT,), _f32),
        pltpu.VMEM((RPT,), _f32),
        pltpu.VMEM_SHARED((NP,), _f32),
    ],
    compiler_params=_sc_params,
)


def _hop_body(g_hbm, src_hbm, dst_hbm, a_out,
              sidx, didx, rows0, rows1, rows2, rows3, acc_sp,
              sem0, sem1, sem2, sem3):
    c = lax.axis_index("c")
    s = lax.axis_index("s")
    bufs = (rows0, rows1, rows2, rows3)
    sems = (sem0, sem1, sem2, sem3)

    pltpu.sync_copy(src_hbm.at[s], sidx)
    pltpu.sync_copy(dst_hbm.at[s], didx)

    def _zero(i, _):
        for k in range(DH // 16):
            rows0[i, pl.ds(k * 16, 16)] = jnp.zeros((16,), _f32)
        return ()

    lax.fori_loop(0, CH, _zero, ())
    for q in range(RPT // CH):
        pltpu.sync_copy(rows0, acc_sp.at[pl.ds(s * RPT + q * CH, CH)])
    plsc.subcore_barrier()

    gsrc = g_hbm.at[c]

    pltpu.async_copy(gsrc.at[sidx.at[0]], rows0, sem0)

    def _pair(jj, _):
        j0 = 2 * jj
        j1 = j0 + 1
        pltpu.make_async_copy(gsrc.at[sidx.at[j0]], rows0, sem0).wait()
        pltpu.async_copy(gsrc.at[sidx.at[j1]], rows1, sem1)
        pltpu.sync_copy(rows0, acc_sp.at[didx.at[j0]], add=True)

        @pl.when(j1 + 1 < NCH)
        def _():
            pltpu.async_copy(gsrc.at[sidx.at[j1 + 1]], rows0, sem0)

        pltpu.make_async_copy(gsrc.at[sidx.at[j1]], rows1, sem1).wait()
        pltpu.sync_copy(rows1, acc_sp.at[didx.at[j1]], add=True)
        return ()

    lax.fori_loop(0, NCH // 2, _pair, ())
    plsc.subcore_barrier()
    pltpu.sync_copy(acc_sp.at[pl.ds(s * RPT, RPT)],
                    a_out.at[c, pl.ds(s * RPT, RPT)])


_hop_call = pl.kernel(
    _hop_body,
    out_type=jax.ShapeDtypeStruct((2, NP, DH), _f32),
    mesh=_mesh,
    scratch_types=[
        pltpu.VMEM((NCH, CH), jnp.int32),
        pltpu.VMEM((NCH, CH), jnp.int32),
        pltpu.VMEM((CH, DH), _f32),
        pltpu.VMEM((CH, DH), _f32),
        pltpu.VMEM((CH, DH), _f32),
        pltpu.VMEM((CH, DH), _f32),
        pltpu.VMEM_SHARED((NP, DH), _f32),
        pltpu.SemaphoreType.DMA,
        pltpu.SemaphoreType.DMA,
        pltpu.SemaphoreType.DMA,
        pltpu.SemaphoreType.DMA,
    ],
    compiler_params=_sc_params,
)


R = 512


def _prep_body(deg_ref, feat_ref, norm_ref, inv_ref, g0_ref):
    d = jnp.maximum(deg_ref[0] + deg_ref[1], 1.0)
    nr = lax.rsqrt(d)
    norm_ref[...] = nr
    inv_ref[...] = nr * nr
    g0_ref[...] = feat_ref[...] * nr[None]


def _tc_prep(deg2, feat2):
    return pl.pallas_call(
        _prep_body,
        grid=(NP // R,),
        in_specs=[
            pl.BlockSpec((2, R, 1), lambda r: (0, r, 0)),
            pl.BlockSpec((2, R, DH), lambda r: (0, r, 0)),
        ],
        out_specs=[
            pl.BlockSpec((R, 1), lambda r: (r, 0)),
            pl.BlockSpec((R, 1), lambda r: (r, 0)),
            pl.BlockSpec((2, R, DH), lambda r: (0, r, 0)),
        ],
        out_shape=[
            jax.ShapeDtypeStruct((NP, 1), _f32),
            jax.ShapeDtypeStruct((NP, 1), _f32),
            jax.ShapeDtypeStruct((2, NP, DH), _f32),
        ],
    )(deg2, feat2)


def _scale_body(inv_ref, a_ref, g_ref):
    g_ref[...] = a_ref[...] * inv_ref[...][None]


def _tc_scale(inv, a1):
    return pl.pallas_call(
        _scale_body,
        grid=(NP // R,),
        in_specs=[
            pl.BlockSpec((R, 1), lambda r: (r, 0)),
            pl.BlockSpec((2, R, DH), lambda r: (0, r, 0)),
        ],
        out_specs=pl.BlockSpec((2, R, DH), lambda r: (0, r, 0)),
        out_shape=jax.ShapeDtypeStruct((2, NP, DH), _f32),
    )(inv, a1)


def _final_body(norm_ref, a2_ref, feat_ref, wfc_ref, wres_ref, b_ref,
                out_ref):
    nr = norm_ref[...]
    h_lo = a2_ref[0] * nr
    h_hi = a2_ref[1] * nr
    acc = jnp.dot(h_lo, wfc_ref[pl.ds(0, DH), :],
                  preferred_element_type=_f32)
    acc += jnp.dot(h_hi, wfc_ref[pl.ds(DH, DH), :],
                   preferred_element_type=_f32)
    acc += jnp.dot(feat_ref[...], wres_ref[...],
                   preferred_element_type=_f32)
    out_ref[...] = acc + b_ref[...]


def _tc_final(norm, a2, feat_pad, W_fc, W_res, bias):
    return pl.pallas_call(
        _final_body,
        grid=(NP // R,),
        in_specs=[
            pl.BlockSpec((R, 1), lambda r: (r, 0)),
            pl.BlockSpec((2, R, DH), lambda r: (0, r, 0)),
            pl.BlockSpec((R, D), lambda r: (r, 0)),
            pl.BlockSpec((D, D), lambda r: (0, 0)),
            pl.BlockSpec((D, D), lambda r: (0, 0)),
            pl.BlockSpec((1, D), lambda r: (0, 0)),
        ],
        out_specs=pl.BlockSpec((R, D), lambda r: (r, 0)),
        out_shape=jax.ShapeDtypeStruct((NP, D), _f32),
    )(norm, a2, feat_pad, W_fc, W_res, bias)


def kernel(feat, edge_index, W_fc, b_fc, W_res, b_res):
    src = edge_index[0]
    dst = edge_index[1]
    pad = jnp.full((ETOT - E,), PAD_IDX, jnp.int32)
    src_p = jnp.concatenate([src, pad])
    dst_p = jnp.concatenate([dst, pad])
    src16 = src_p.reshape(NS, NCH, CH)
    dst16 = dst_p.reshape(NS, NCH, CH)
    dst32 = dst_p.reshape(2, NS, NCH32, CH)

    feat_pad = jnp.pad(feat, ((0, NP - N), (0, 0)))
    feat2 = feat_pad.reshape(NP, 2, DH).transpose(1, 0, 2)

    deg2 = _deg_call(dst32)
    norm, inv, g0 = _tc_prep(deg2[..., None], feat2)
    a1 = _hop_call(g0, src16, dst16)
    g1 = _tc_scale(inv, a1)
    a2 = _hop_call(g1, src16, dst16)
    bias = (b_fc + b_res)[None, :]
    out_pad = _tc_final(norm, a2, feat_pad, W_fc, W_res, bias)
    return out_pad[:N]


# speedup vs baseline: 2.0060x; 2.0060x over previous
import jax
import jax.numpy as jnp
from jax import lax
from jax.experimental import pallas as pl
from jax.experimental.pallas import tpu as pltpu
from jax.experimental.pallas import tpu_sc as plsc

N = 10000
D = 128
DH = 64
NP = 10240
NS = 16
RPT = NP // NS
CH = 128
E = 320000
KB = 2
NCH32 = 80
ETOT = 32 * NCH32 * CH
NCH = 2 * NCH32
NBLK = NCH // KB
PAD_IDX = NP - 1

_mesh = plsc.VectorSubcoreMesh(core_axis_name="c", subcore_axis_name="s")

_f32 = jnp.float32

_sc_params = pltpu.CompilerParams(use_tc_tiling_on_sc=False)


def _deg_body(dst_hbm, deg_out, idx_v, ones_v, zbuf, deg_sp):
    c = lax.axis_index("c")
    s = lax.axis_index("s")

    def _fill(i, _):
        ones_v[pl.ds(i * 16, 16)] = jnp.ones((16,), _f32)
        zbuf[pl.ds(i * 16, 16)] = jnp.zeros((16,), _f32)
        return ()

    lax.fori_loop(0, RPT // 16, _fill, ())

    pltpu.sync_copy(dst_hbm.at[c, s], idx_v)
    pltpu.sync_copy(zbuf, deg_sp.at[pl.ds(s * RPT, RPT)])
    plsc.subcore_barrier()

    def _chunk(j, _):
        pltpu.sync_copy(ones_v.at[pl.ds(0, CH)], deg_sp.at[idx_v.at[j]],
                        add=True)
        return ()

    lax.fori_loop(0, NCH32, _chunk, ())
    plsc.subcore_barrier()

    @pl.when(s == 0)
    def _():
        pltpu.sync_copy(deg_sp, deg_out.at[c])


_deg_call = pl.kernel(
    _deg_body,
    out_type=jax.ShapeDtypeStruct((2, NP), _f32),
    mesh=_mesh,
    scratch_types=[
        pltpu.VMEM((NCH32, CH), jnp.int32),
        pltpu.VMEM((RPT,), _f32),
        pltpu.VMEM((RPT,), _f32),
        pltpu.VMEM_SHARED((NP,), _f32),
    ],
    compiler_params=_sc_params,
)


def _hop_body(g_hbm, src_hbm, dst_hbm, a_out,
              sidx, didx, rows0, rows1, rows2, rows3, acc_sp,
              sem0, sem1, sem2, sem3):
    c = lax.axis_index("c")
    s = lax.axis_index("s")
    bufs = (rows0, rows1, rows2, rows3)
    sems = (sem0, sem1, sem2, sem3)

    pltpu.sync_copy(src_hbm.at[s], sidx)
    pltpu.sync_copy(dst_hbm.at[s], didx)

    def _zero(i, _):
        for k in range(DH // 16):
            rows0[i, pl.ds(k * 16, 16)] = jnp.zeros((16,), _f32)
        return ()

    lax.fori_loop(0, CH, _zero, ())
    for q in range(RPT // CH):
        pltpu.sync_copy(rows0, acc_sp.at[pl.ds(s * RPT + q * CH, CH)])
    plsc.subcore_barrier()

    gsrc = g_hbm.at[c]

    pltpu.async_copy(gsrc.at[sidx.at[0]], rows0, sem0)

    def _pair(jj, _):
        j0 = 2 * jj
        j1 = j0 + 1
        pltpu.make_async_copy(gsrc.at[sidx.at[j0]], rows0, sem0).wait()
        pltpu.async_copy(gsrc.at[sidx.at[j1]], rows1, sem1)
        pltpu.sync_copy(rows0, acc_sp.at[didx.at[j0]], add=True)

        @pl.when(j1 + 1 < NCH)
        def _():
            pltpu.async_copy(gsrc.at[sidx.at[j1 + 1]], rows0, sem0)

        pltpu.make_async_copy(gsrc.at[sidx.at[j1]], rows1, sem1).wait()
        pltpu.sync_copy(rows1, acc_sp.at[didx.at[j1]], add=True)
        return ()

    lax.fori_loop(0, NCH // 2, _pair, ())
    plsc.subcore_barrier()
    pltpu.sync_copy(acc_sp.at[pl.ds(s * RPT, RPT)],
                    a_out.at[c, pl.ds(s * RPT, RPT)])


_hop_call = pl.kernel(
    _hop_body,
    out_type=jax.ShapeDtypeStruct((2, NP, DH), _f32),
    mesh=_mesh,
    scratch_types=[
        pltpu.VMEM((NCH, CH), jnp.int32),
        pltpu.VMEM((NCH, CH), jnp.int32),
        pltpu.VMEM((CH, DH), _f32),
        pltpu.VMEM((CH, DH), _f32),
        pltpu.VMEM((CH, DH), _f32),
        pltpu.VMEM((CH, DH), _f32),
        pltpu.VMEM_SHARED((NP, DH), _f32),
        pltpu.SemaphoreType.DMA,
        pltpu.SemaphoreType.DMA,
        pltpu.SemaphoreType.DMA,
        pltpu.SemaphoreType.DMA,
    ],
    compiler_params=_sc_params,
)


R = 512


def _prep_body(deg_ref, feat_ref, norm_ref, inv_ref, g0_ref):
    d = jnp.maximum(deg_ref[0] + deg_ref[1], 1.0)
    nr = lax.rsqrt(d)
    norm_ref[...] = nr
    inv_ref[...] = nr * nr
    g0_ref[...] = feat_ref[...] * nr[None]


def _tc_prep(deg2, feat2):
    return pl.pallas_call(
        _prep_body,
        grid=(NP // R,),
        in_specs=[
            pl.BlockSpec((2, R, 1), lambda r: (0, r, 0)),
            pl.BlockSpec((2, R, DH), lambda r: (0, r, 0)),
        ],
        out_specs=[
            pl.BlockSpec((R, 1), lambda r: (r, 0)),
            pl.BlockSpec((R, 1), lambda r: (r, 0)),
            pl.BlockSpec((2, R, DH), lambda r: (0, r, 0)),
        ],
        out_shape=[
            jax.ShapeDtypeStruct((NP, 1), _f32),
            jax.ShapeDtypeStruct((NP, 1), _f32),
            jax.ShapeDtypeStruct((2, NP, DH), _f32),
        ],
    )(deg2, feat2)


def _scale_body(inv_ref, a_ref, g_ref):
    g_ref[...] = a_ref[...] * inv_ref[...][None]


def _tc_scale(inv, a1):
    return pl.pallas_call(
        _scale_body,
        grid=(NP // R,),
        in_specs=[
            pl.BlockSpec((R, 1), lambda r: (r, 0)),
            pl.BlockSpec((2, R, DH), lambda r: (0, r, 0)),
        ],
        out_specs=pl.BlockSpec((2, R, DH), lambda r: (0, r, 0)),
        out_shape=jax.ShapeDtypeStruct((2, NP, DH), _f32),
    )(inv, a1)


def _final_body(norm_ref, a2_ref, feat_ref, wfc_ref, wres_ref, b_ref,
                out_ref):
    nr = norm_ref[...]
    h_lo = a2_ref[0] * nr
    h_hi = a2_ref[1] * nr
    acc = jnp.dot(h_lo, wfc_ref[pl.ds(0, DH), :],
                  preferred_element_type=_f32)
    acc += jnp.dot(h_hi, wfc_ref[pl.ds(DH, DH), :],
                   preferred_element_type=_f32)
    acc += jnp.dot(feat_ref[...], wres_ref[...],
                   preferred_element_type=_f32)
    out_ref[...] = acc + b_ref[...]


def _tc_final(norm, a2, feat_pad, W_fc, W_res, bias):
    return pl.pallas_call(
        _final_body,
        grid=(NP // R,),
        in_specs=[
            pl.BlockSpec((R, 1), lambda r: (r, 0)),
            pl.BlockSpec((2, R, DH), lambda r: (0, r, 0)),
            pl.BlockSpec((R, D), lambda r: (r, 0)),
            pl.BlockSpec((D, D), lambda r: (0, 0)),
            pl.BlockSpec((D, D), lambda r: (0, 0)),
            pl.BlockSpec((1, D), lambda r: (0, 0)),
        ],
        out_specs=pl.BlockSpec((R, D), lambda r: (r, 0)),
        out_shape=jax.ShapeDtypeStruct((NP, D), _f32),
    )(norm, a2, feat_pad, W_fc, W_res, bias)


def kernel(feat, edge_index, W_fc, b_fc, W_res, b_res):
    src = edge_index[0]
    dst = edge_index[1]
    pad = (N + jnp.arange(ETOT - E, dtype=jnp.int32) % (NP - N)).astype(
        jnp.int32)
    src_p = jnp.concatenate([src, pad])
    dst_p = jnp.concatenate([dst, pad])
    src16 = src_p.reshape(NS, NCH, CH)
    dst16 = dst_p.reshape(NS, NCH, CH)
    dst32 = dst_p.reshape(2, NS, NCH32, CH)

    feat_pad = jnp.pad(feat, ((0, NP - N), (0, 0)))
    feat2 = feat_pad.reshape(NP, 2, DH).transpose(1, 0, 2)

    deg2 = _deg_call(dst32)
    norm, inv, g0 = _tc_prep(deg2[..., None], feat2)
    a1 = _hop_call(g0, src16, dst16)
    g1 = _tc_scale(inv, a1)
    a2 = _hop_call(g1, src16, dst16)
    bias = (b_fc + b_res)[None, :]
    out_pad = _tc_final(norm, a2, feat_pad, W_fc, W_res, bias)
    return out_pad[:N]


# speedup vs baseline: 2.1655x; 1.0795x over previous
import jax
import jax.numpy as jnp
from jax import lax
from jax.experimental import pallas as pl
from jax.experimental.pallas import tpu as pltpu
from jax.experimental.pallas import tpu_sc as plsc

N = 10000
D = 128
DH = 64
NP = 10240
NS = 16
RPT = NP // NS
CH = 128
E = 320000
NCH = 160
ETOT = NS * NCH * CH
QR = RPT // CH

_mesh = plsc.VectorSubcoreMesh(core_axis_name="c", subcore_axis_name="s")

_f32 = jnp.float32

_sc_params = pltpu.CompilerParams(use_tc_tiling_on_sc=False,
                                  needs_layout_passes=False)


def _sc_body(feat2, src_hbm, dst_hbm, a_out, norm_out, g_hbm,
             sidx, didx, rows0, rows1, zbuf, fbuf,
             onesv, degv, normv, invv, zvec, acc_sp, deg_sp, sem0, sem1):
    c = lax.axis_index("c")
    s = lax.axis_index("s")
    r0 = s * RPT

    pltpu.sync_copy(src_hbm.at[s], sidx)
    pltpu.sync_copy(dst_hbm.at[s], didx)

    def _fill_zb(i, _):
        for k in range(DH // 16):
            zbuf[i, pl.ds(k * 16, 16)] = jnp.zeros((16,), _f32)
        return ()

    lax.fori_loop(0, CH, _fill_zb, ())
    for k in range(CH // 16):
        onesv[pl.ds(k * 16, 16)] = jnp.ones((16,), _f32)
    for k in range(RPT // 16):
        zvec[pl.ds(k * 16, 16)] = jnp.zeros((16,), _f32)

    pltpu.sync_copy(zvec, deg_sp.at[pl.ds(r0, RPT)])
    plsc.subcore_barrier()

    def _dchunk(j, _):
        pltpu.sync_copy(onesv, deg_sp.at[didx.at[j]], add=True)
        return ()

    lax.fori_loop(0, NCH, _dchunk, ())
    plsc.subcore_barrier()

    pltpu.sync_copy(deg_sp.at[pl.ds(r0, RPT)], degv)

    def _newton(i, _):
        sl = pl.ds(i * 16, 16)
        d = jnp.maximum(degv[sl], 1.0)
        bits = plsc.bitcast(d, jnp.int32)
        y = plsc.bitcast(jnp.int32(0x5F3759DF) - (bits >> 1), _f32)
        for _it in range(3):
            y = y * (1.5 - 0.5 * d * y * y)
        normv[sl] = y
        invv[sl] = y * y
        return ()

    lax.fori_loop(0, RPT // 16, _newton, ())

    @pl.when(c == 0)
    def _():
        pltpu.sync_copy(normv, norm_out.at[pl.ds(r0, RPT)])

    def _scale_fbuf(q, scale_ref):
        def _sgrp(i, _):
            nv = scale_ref[pl.ds(q * CH + i * 16, 16)]
            for r in range(16):
                vb = nv[jnp.full((16,), r, jnp.int32)]
                for k in range(DH // 16):
                    sl = pl.ds(k * 16, 16)
                    fbuf[i * 16 + r, sl] = fbuf[i * 16 + r, sl] * vb
            return ()

        lax.fori_loop(0, CH // 16, _sgrp, ())

    for q in range(QR):
        pltpu.sync_copy(feat2.at[c, pl.ds(r0 + q * CH, CH)], fbuf)
        _scale_fbuf(q, normv)
        pltpu.sync_copy(fbuf, g_hbm.at[c, pl.ds(r0 + q * CH, CH)])
    for q in range(QR):
        pltpu.sync_copy(zbuf, acc_sp.at[pl.ds(r0 + q * CH, CH)])
    plsc.subcore_barrier()

    gsrc = g_hbm.at[c]

    def _hop():
        pltpu.async_copy(gsrc.at[sidx.at[0]], rows0, sem0)

        def _pair(jj, _):
            j0 = 2 * jj
            j1 = j0 + 1
            pltpu.make_async_copy(gsrc.at[sidx.at[j0]], rows0, sem0).wait()
            pltpu.async_copy(gsrc.at[sidx.at[j1]], rows1, sem1)
            pltpu.sync_copy(rows0, acc_sp.at[didx.at[j0]], add=True)

            @pl.when(j1 + 1 < NCH)
            def _():
                pltpu.async_copy(gsrc.at[sidx.at[j1 + 1]], rows0, sem0)

            pltpu.make_async_copy(gsrc.at[sidx.at[j1]], rows1, sem1).wait()
            pltpu.sync_copy(rows1, acc_sp.at[didx.at[j1]], add=True)
            return ()

        lax.fori_loop(0, NCH // 2, _pair, ())
        plsc.subcore_barrier()

    _hop()

    def _a_rows(q):
        return acc_sp.at[pl.ds(r0 + q * CH, CH)]

    for q in range(QR):
        pltpu.sync_copy(_a_rows(q), fbuf)
        _scale_fbuf(q, invv)
        pltpu.sync_copy(fbuf, g_hbm.at[c, pl.ds(r0 + q * CH, CH)])
        pltpu.sync_copy(zbuf, _a_rows(q))
    plsc.subcore_barrier()

    _hop()

    pltpu.sync_copy(acc_sp.at[pl.ds(r0, RPT)],
                    a_out.at[c, pl.ds(r0, RPT)])


_sc_call = pl.kernel(
    _sc_body,
    out_type=(
        jax.ShapeDtypeStruct((2, NP, DH), _f32),
        jax.ShapeDtypeStruct((NP,), _f32),
        jax.ShapeDtypeStruct((2, NP, DH), _f32),
    ),
    mesh=_mesh,
    scratch_types=[
        pltpu.VMEM((NCH, CH), jnp.int32),
        pltpu.VMEM((NCH, CH), jnp.int32),
        pltpu.VMEM((CH, DH), _f32),
        pltpu.VMEM((CH, DH), _f32),
        pltpu.VMEM((CH, DH), _f32),
        pltpu.VMEM((CH, DH), _f32),
        pltpu.VMEM((CH,), _f32),
        pltpu.VMEM((RPT,), _f32),
        pltpu.VMEM((RPT,), _f32),
        pltpu.VMEM((RPT,), _f32),
        pltpu.VMEM((RPT,), _f32),
        pltpu.VMEM_SHARED((NP, DH), _f32),
        pltpu.VMEM_SHARED((NP,), _f32),
        pltpu.SemaphoreType.DMA,
        pltpu.SemaphoreType.DMA,
    ],
    compiler_params=_sc_params,
)


R = 512


def _final_body(norm_ref, a2_ref, feat_ref, wfc_ref, wres_ref, b_ref,
                out_ref):
    nr = norm_ref[...]
    h_lo = a2_ref[0] * nr
    h_hi = a2_ref[1] * nr
    acc = jnp.dot(h_lo, wfc_ref[pl.ds(0, DH), :],
                  preferred_element_type=_f32)
    acc += jnp.dot(h_hi, wfc_ref[pl.ds(DH, DH), :],
                   preferred_element_type=_f32)
    acc += jnp.dot(feat_ref[...], wres_ref[...],
                   preferred_element_type=_f32)
    out_ref[...] = acc + b_ref[...]


def _tc_final(norm, a2, feat_pad, W_fc, W_res, bias):
    return pl.pallas_call(
        _final_body,
        grid=(NP // R,),
        in_specs=[
            pl.BlockSpec((R, 1), lambda r: (r, 0)),
            pl.BlockSpec((2, R, DH), lambda r: (0, r, 0)),
            pl.BlockSpec((R, D), lambda r: (r, 0)),
            pl.BlockSpec((D, D), lambda r: (0, 0)),
            pl.BlockSpec((D, D), lambda r: (0, 0)),
            pl.BlockSpec((1, D), lambda r: (0, 0)),
        ],
        out_specs=pl.BlockSpec((R, D), lambda r: (r, 0)),
        out_shape=jax.ShapeDtypeStruct((NP, D), _f32),
    )(norm, a2, feat_pad, W_fc, W_res, bias)


def kernel(feat, edge_index, W_fc, b_fc, W_res, b_res):
    src = edge_index[0]
    dst = edge_index[1]
    pad = (N + jnp.arange(ETOT - E, dtype=jnp.int32) % (NP - N)).astype(
        jnp.int32)
    src16 = jnp.concatenate([src, pad]).reshape(NS, NCH, CH)
    dst16 = jnp.concatenate([dst, pad]).reshape(NS, NCH, CH)

    feat_pad = jnp.pad(feat, ((0, NP - N), (0, 0)))
    feat2 = feat_pad.reshape(NP, 2, DH).transpose(1, 0, 2)

    a2, norm, _ = _sc_call(feat2, src16, dst16)
    bias = (b_fc + b_res)[None, :]
    out_pad = _tc_final(norm[:, None], a2, feat_pad, W_fc, W_res, bias)
    return out_pad[:N]


# speedup vs baseline: 2.3944x; 1.1057x over previous
import jax
import jax.numpy as jnp
from jax import lax
from jax.experimental import pallas as pl
from jax.experimental.pallas import tpu as pltpu
from jax.experimental.pallas import tpu_sc as plsc

N = 10000
D = 128
DH = 64
NP = 10240
NS = 16
RPT = NP // NS
CH = 128
E = 320000
NCHT = E // CH
CPT = NCHT // NS
XTR = NCHT - NS * CPT
QR = RPT // CH

_mesh = plsc.VectorSubcoreMesh(core_axis_name="c", subcore_axis_name="s")

_f32 = jnp.float32

_sc_params = pltpu.CompilerParams(use_tc_tiling_on_sc=False,
                                  needs_layout_passes=False)


def _sc_body(featp, e3, a_out, norm_out, g_hbm,
             sidx, didx, rows0, rows1, zbuf, fbuf,
             onesv, degv, normv, invv, zvec, acc_sp, deg_sp, sem0, sem1):
    c = lax.axis_index("c")
    s = lax.axis_index("s")
    r0 = s * RPT
    start = s * CPT + jnp.minimum(s, XTR)
    has_extra = s < XTR

    pltpu.sync_copy(e3.at[0, pl.ds(start, CPT)], sidx.at[pl.ds(0, CPT)])
    pltpu.sync_copy(e3.at[1, pl.ds(start, CPT)], didx.at[pl.ds(0, CPT)])

    @pl.when(has_extra)
    def _():
        pltpu.sync_copy(e3.at[0, start + CPT], sidx.at[CPT])
        pltpu.sync_copy(e3.at[1, start + CPT], didx.at[CPT])

    def _fill_zb(i, _):
        for k in range(DH // 16):
            zbuf[i, pl.ds(k * 16, 16)] = jnp.zeros((16,), _f32)
        return ()

    lax.fori_loop(0, CH, _fill_zb, ())
    for k in range(CH // 16):
        onesv[pl.ds(k * 16, 16)] = jnp.ones((16,), _f32)
    for k in range(RPT // 16):
        zvec[pl.ds(k * 16, 16)] = jnp.zeros((16,), _f32)

    pltpu.sync_copy(zvec, deg_sp.at[pl.ds(r0, RPT)])
    plsc.subcore_barrier()

    def _dchunk(j, _):
        pltpu.sync_copy(onesv, deg_sp.at[didx.at[j]], add=True)
        return ()

    lax.fori_loop(0, CPT, _dchunk, ())

    @pl.when(has_extra)
    def _():
        pltpu.sync_copy(onesv, deg_sp.at[didx.at[CPT]], add=True)

    plsc.subcore_barrier()

    pltpu.sync_copy(deg_sp.at[pl.ds(r0, RPT)], degv)

    def _newton(i, _):
        sl = pl.ds(i * 16, 16)
        d = jnp.maximum(degv[sl], 1.0)
        bits = plsc.bitcast(d, jnp.int32)
        y = plsc.bitcast(jnp.int32(0x5F3759DF) - (bits >> 1), _f32)
        for _it in range(3):
            y = y * (1.5 - 0.5 * d * y * y)
        normv[sl] = y
        invv[sl] = y * y
        return ()

    lax.fori_loop(0, RPT // 16, _newton, ())

    @pl.when(c == 0)
    def _():
        pltpu.sync_copy(normv, norm_out.at[pl.ds(r0, RPT)])

    def _scale_fbuf(q, scale_ref):
        def _sgrp(i, _):
            nv = scale_ref[pl.ds(q * CH + i * 16, 16)]
            for r in range(16):
                vb = nv[jnp.full((16,), r, jnp.int32)]
                for k in range(DH // 16):
                    sl = pl.ds(k * 16, 16)
                    fbuf[i * 16 + r, sl] = fbuf[i * 16 + r, sl] * vb
            return ()

        lax.fori_loop(0, CH // 16, _sgrp, ())

    for q in range(QR):
        pltpu.sync_copy(
            featp.at[pl.ds(r0 + q * CH, CH), pl.ds(c * DH, DH)], fbuf)
        _scale_fbuf(q, normv)
        pltpu.sync_copy(fbuf, g_hbm.at[c, pl.ds(r0 + q * CH, CH)])
    for q in range(QR):
        pltpu.sync_copy(zbuf, acc_sp.at[pl.ds(r0 + q * CH, CH)])
    plsc.subcore_barrier()

    gsrc = g_hbm.at[c]

    def _hop():
        pltpu.async_copy(gsrc.at[sidx.at[0]], rows0, sem0)

        def _pair(jj, _):
            j0 = 2 * jj
            j1 = j0 + 1
            pltpu.make_async_copy(gsrc.at[sidx.at[j0]], rows0, sem0).wait()
            pltpu.async_copy(gsrc.at[sidx.at[j1]], rows1, sem1)
            pltpu.sync_copy(rows0, acc_sp.at[didx.at[j0]], add=True)

            @pl.when(j1 + 1 < CPT)
            def _():
                pltpu.async_copy(gsrc.at[sidx.at[j1 + 1]], rows0, sem0)

            pltpu.make_async_copy(gsrc.at[sidx.at[j1]], rows1, sem1).wait()
            pltpu.sync_copy(rows1, acc_sp.at[didx.at[j1]], add=True)
            return ()

        lax.fori_loop(0, CPT // 2, _pair, ())

        @pl.when(has_extra)
        def _():
            pltpu.sync_copy(gsrc.at[sidx.at[CPT]], rows0)
            pltpu.sync_copy(rows0, acc_sp.at[didx.at[CPT]], add=True)

        plsc.subcore_barrier()

    _hop()

    def _a_rows(q):
        return acc_sp.at[pl.ds(r0 + q * CH, CH)]

    for q in range(QR):
        pltpu.sync_copy(_a_rows(q), fbuf)
        _scale_fbuf(q, invv)
        pltpu.sync_copy(fbuf, g_hbm.at[c, pl.ds(r0 + q * CH, CH)])
        pltpu.sync_copy(zbuf, _a_rows(q))
    plsc.subcore_barrier()

    _hop()

    pltpu.sync_copy(acc_sp.at[pl.ds(r0, RPT)],
                    a_out.at[c, pl.ds(r0, RPT)])


_sc_call = pl.kernel(
    _sc_body,
    out_type=(
        jax.ShapeDtypeStruct((2, NP, DH), _f32),
        jax.ShapeDtypeStruct((NP,), _f32),
        jax.ShapeDtypeStruct((2, NP, DH), _f32),
    ),
    mesh=_mesh,
    scratch_types=[
        pltpu.VMEM((CPT + 1, CH), jnp.int32),
        pltpu.VMEM((CPT + 1, CH), jnp.int32),
        pltpu.VMEM((CH, DH), _f32),
        pltpu.VMEM((CH, DH), _f32),
        pltpu.VMEM((CH, DH), _f32),
        pltpu.VMEM((CH, DH), _f32),
        pltpu.VMEM((CH,), _f32),
        pltpu.VMEM((RPT,), _f32),
        pltpu.VMEM((RPT,), _f32),
        pltpu.VMEM((RPT,), _f32),
        pltpu.VMEM((RPT,), _f32),
        pltpu.VMEM_SHARED((NP, DH), _f32),
        pltpu.VMEM_SHARED((NP,), _f32),
        pltpu.SemaphoreType.DMA,
        pltpu.SemaphoreType.DMA,
    ],
    compiler_params=_sc_params,
)


R = 512


def _final_body(norm_ref, a2_ref, feat_ref, wfc_ref, wres_ref, b_ref,
                out_ref):
    nr = norm_ref[...]
    h_lo = a2_ref[0] * nr
    h_hi = a2_ref[1] * nr
    acc = jnp.dot(h_lo, wfc_ref[pl.ds(0, DH), :],
                  preferred_element_type=_f32)
    acc += jnp.dot(h_hi, wfc_ref[pl.ds(DH, DH), :],
                   preferred_element_type=_f32)
    acc += jnp.dot(feat_ref[...], wres_ref[...],
                   preferred_element_type=_f32)
    out_ref[...] = acc + b_ref[...]


def _tc_final(norm, a2, feat_pad, W_fc, W_res, bias):
    return pl.pallas_call(
        _final_body,
        grid=(NP // R,),
        in_specs=[
            pl.BlockSpec((R, 1), lambda r: (r, 0)),
            pl.BlockSpec((2, R, DH), lambda r: (0, r, 0)),
            pl.BlockSpec((R, D), lambda r: (r, 0)),
            pl.BlockSpec((D, D), lambda r: (0, 0)),
            pl.BlockSpec((D, D), lambda r: (0, 0)),
            pl.BlockSpec((1, D), lambda r: (0, 0)),
        ],
        out_specs=pl.BlockSpec((R, D), lambda r: (r, 0)),
        out_shape=jax.ShapeDtypeStruct((N, D), _f32),
    )(norm, a2, feat_pad, W_fc, W_res, bias)


def kernel(feat, edge_index, W_fc, b_fc, W_res, b_res):
    e3 = edge_index.reshape(2, NCHT, CH)
    feat_pad = jnp.pad(feat, ((0, NP - N), (0, 0)))

    a2, norm, _ = _sc_call(feat_pad, e3)
    bias = (b_fc + b_res)[None, :]
    return _tc_final(norm[:, None], a2, feat_pad, W_fc, W_res, bias)


# speedup vs baseline: 2.3979x; 1.0015x over previous
import jax
import jax.numpy as jnp
from jax import lax
from jax.experimental import pallas as pl
from jax.experimental.pallas import tpu as pltpu
from jax.experimental.pallas import tpu_sc as plsc

N = 10000
D = 128
DH = 64
NP = 10240
NS = 16
RPT = NP // NS
CH = 128
E = 320000
NCHT = E // CH
CPT = NCHT // NS
XTR = NCHT - NS * CPT
QR = RPT // CH

_mesh = plsc.VectorSubcoreMesh(core_axis_name="c", subcore_axis_name="s")

_f32 = jnp.float32

_sc_params = pltpu.CompilerParams(use_tc_tiling_on_sc=False,
                                  needs_layout_passes=False)


def _sc_body(featp, e3, a_out, norm_out, g_hbm,
             sidx, didx, rows0, rows1, zbuf, fbuf,
             onesv, degv, normv, invv, zvec, acc_sp, deg_sp, sem0, sem1):
    c = lax.axis_index("c")
    s = lax.axis_index("s")
    r0 = s * RPT
    start = s * CPT + jnp.minimum(s, XTR)
    has_extra = s < XTR

    pltpu.sync_copy(e3.at[0, pl.ds(start, CPT)], sidx.at[pl.ds(0, CPT)])
    pltpu.sync_copy(e3.at[1, pl.ds(start, CPT)], didx.at[pl.ds(0, CPT)])

    @pl.when(has_extra)
    def _():
        pltpu.sync_copy(e3.at[0, start + CPT], sidx.at[CPT])
        pltpu.sync_copy(e3.at[1, start + CPT], didx.at[CPT])

    def _fill_zb(i, _):
        for k in range(DH // 16):
            zbuf[i, pl.ds(k * 16, 16)] = jnp.zeros((16,), _f32)
        return ()

    lax.fori_loop(0, CH, _fill_zb, ())
    for k in range(CH // 16):
        onesv[pl.ds(k * 16, 16)] = jnp.ones((16,), _f32)
    for k in range(RPT // 16):
        zvec[pl.ds(k * 16, 16)] = jnp.zeros((16,), _f32)

    pltpu.sync_copy(zvec, deg_sp.at[pl.ds(r0, RPT)])
    plsc.subcore_barrier()

    def _dchunk(j, _):
        pltpu.sync_copy(onesv, deg_sp.at[didx.at[j]], add=True)
        return ()

    lax.fori_loop(0, CPT, _dchunk, ())

    @pl.when(has_extra)
    def _():
        pltpu.sync_copy(onesv, deg_sp.at[didx.at[CPT]], add=True)

    plsc.subcore_barrier()

    pltpu.sync_copy(deg_sp.at[pl.ds(r0, RPT)], degv)

    def _newton(i, _):
        sl = pl.ds(i * 16, 16)
        d = jnp.maximum(degv[sl], 1.0)
        bits = plsc.bitcast(d, jnp.int32)
        y = plsc.bitcast(jnp.int32(0x5F3759DF) - (bits >> 1), _f32)
        for _it in range(3):
            y = y * (1.5 - 0.5 * d * y * y)
        normv[sl] = y
        invv[sl] = y * y
        return ()

    lax.fori_loop(0, RPT // 16, _newton, ())

    @pl.when(c == 0)
    def _():
        pltpu.sync_copy(normv, norm_out.at[pl.ds(r0, RPT)])

    def _scale_fbuf(q, scale_ref):
        def _sgrp(i, _):
            nv = scale_ref[pl.ds(q * CH + i * 16, 16)]
            for r in range(16):
                vb = nv[jnp.full((16,), r, jnp.int32)]
                for k in range(DH // 16):
                    sl = pl.ds(k * 16, 16)
                    fbuf[i * 16 + r, sl] = fbuf[i * 16 + r, sl] * vb
            return ()

        lax.fori_loop(0, CH // 16, _sgrp, ())

    for q in range(QR):
        pltpu.sync_copy(
            featp.at[pl.ds(r0 + q * CH, CH), pl.ds(c * DH, DH)], fbuf)
        _scale_fbuf(q, normv)
        pltpu.sync_copy(fbuf, g_hbm.at[c, pl.ds(r0 + q * CH, CH)])
    for q in range(QR):
        pltpu.sync_copy(zbuf, acc_sp.at[pl.ds(r0 + q * CH, CH)])
    plsc.subcore_barrier()

    gsrc = g_hbm.at[c]

    def _hop():
        pltpu.async_copy(gsrc.at[sidx.at[0]], rows0, sem0)

        def _pair(jj, _):
            j0 = 2 * jj
            j1 = j0 + 1
            pltpu.make_async_copy(gsrc.at[sidx.at[j0]], rows0, sem0).wait()
            pltpu.async_copy(gsrc.at[sidx.at[j1]], rows1, sem1)
            pltpu.sync_copy(rows0, acc_sp.at[didx.at[j0]], add=True)

            @pl.when(j1 + 1 < CPT)
            def _():
                pltpu.async_copy(gsrc.at[sidx.at[j1 + 1]], rows0, sem0)

            pltpu.make_async_copy(gsrc.at[sidx.at[j1]], rows1, sem1).wait()
            pltpu.sync_copy(rows1, acc_sp.at[didx.at[j1]], add=True)
            return ()

        lax.fori_loop(0, CPT // 2, _pair, ())

        @pl.when(has_extra)
        def _():
            pltpu.sync_copy(gsrc.at[sidx.at[CPT]], rows0)
            pltpu.sync_copy(rows0, acc_sp.at[didx.at[CPT]], add=True)

        plsc.subcore_barrier()

    _hop()

    def _a_rows(q):
        return acc_sp.at[pl.ds(r0 + q * CH, CH)]

    for q in range(QR):
        pltpu.sync_copy(_a_rows(q), fbuf)
        _scale_fbuf(q, invv)
        pltpu.sync_copy(fbuf, g_hbm.at[c, pl.ds(r0 + q * CH, CH)])
        pltpu.sync_copy(zbuf, _a_rows(q))
    plsc.subcore_barrier()

    _hop()

    pltpu.sync_copy(acc_sp.at[pl.ds(r0, RPT)],
                    a_out.at[c, pl.ds(r0, RPT)])


_sc_call = pl.kernel(
    _sc_body,
    out_type=(
        jax.ShapeDtypeStruct((2, NP, DH), _f32),
        jax.ShapeDtypeStruct((NP,), _f32),
        jax.ShapeDtypeStruct((2, NP, DH), _f32),
    ),
    mesh=_mesh,
    scratch_types=[
        pltpu.VMEM((CPT + 1, CH), jnp.int32),
        pltpu.VMEM((CPT + 1, CH), jnp.int32),
        pltpu.VMEM((CH, DH), _f32),
        pltpu.VMEM((CH, DH), _f32),
        pltpu.VMEM((CH, DH), _f32),
        pltpu.VMEM((CH, DH), _f32),
        pltpu.VMEM((CH,), _f32),
        pltpu.VMEM((RPT,), _f32),
        pltpu.VMEM((RPT,), _f32),
        pltpu.VMEM((RPT,), _f32),
        pltpu.VMEM((RPT,), _f32),
        pltpu.VMEM_SHARED((NP, DH), _f32),
        pltpu.VMEM_SHARED((NP,), _f32),
        pltpu.SemaphoreType.DMA,
        pltpu.SemaphoreType.DMA,
    ],
    compiler_params=_sc_params,
)


R = 512


def _res_body(feat_ref, wres_ref, b_ref, out_ref):
    out_ref[...] = jnp.dot(feat_ref[...], wres_ref[...],
                           preferred_element_type=_f32) + b_ref[...]


def _tc_res(feat, W_res, bias):
    return pl.pallas_call(
        _res_body,
        grid=(NP // R,),
        in_specs=[
            pl.BlockSpec((R, D), lambda r: (r, 0)),
            pl.BlockSpec((D, D), lambda r: (0, 0)),
            pl.BlockSpec((1, D), lambda r: (0, 0)),
        ],
        out_specs=pl.BlockSpec((R, D), lambda r: (r, 0)),
        out_shape=jax.ShapeDtypeStruct((N, D), _f32),
    )(feat, W_res, bias)


def _final_body(norm_ref, a2_ref, res_ref, wfc_ref, out_ref):
    nr = norm_ref[...]
    h_lo = a2_ref[0] * nr
    h_hi = a2_ref[1] * nr
    acc = jnp.dot(h_lo, wfc_ref[pl.ds(0, DH), :],
                  preferred_element_type=_f32)
    acc += jnp.dot(h_hi, wfc_ref[pl.ds(DH, DH), :],
                   preferred_element_type=_f32)
    out_ref[...] = acc + res_ref[...]


def _tc_final(norm, a2, res, W_fc):
    return pl.pallas_call(
        _final_body,
        grid=(NP // R,),
        in_specs=[
            pl.BlockSpec((R, 1), lambda r: (r, 0)),
            pl.BlockSpec((2, R, DH), lambda r: (0, r, 0)),
            pl.BlockSpec((R, D), lambda r: (r, 0)),
            pl.BlockSpec((D, D), lambda r: (0, 0)),
        ],
        out_specs=pl.BlockSpec((R, D), lambda r: (r, 0)),
        out_shape=jax.ShapeDtypeStruct((N, D), _f32),
    )(norm, a2, res, W_fc)


def kernel(feat, edge_index, W_fc, b_fc, W_res, b_res):
    e3 = edge_index.reshape(2, NCHT, CH)
    feat_pad = jnp.pad(feat, ((0, NP - N), (0, 0)))

    bias = (b_fc + b_res)[None, :]
    res = _tc_res(feat, W_res, bias)
    a2, norm, _ = _sc_call(feat_pad, e3)
    return _tc_final(norm[:, None], a2, res, W_fc)


# speedup vs baseline: 2.9826x; 1.2439x over previous
import jax
import jax.numpy as jnp
from jax import lax
from jax.experimental import pallas as pl
from jax.experimental.pallas import tpu as pltpu
from jax.experimental.pallas import tpu_sc as plsc

N = 10000
D = 128
DH = 64
NP = 10240
NS = 16
RPT = NP // NS
CH = 128
E = 320000
NCHT = E // CH
CPT = NCHT // NS
XTR = NCHT - NS * CPT
QR = RPT // CH

_mesh = plsc.VectorSubcoreMesh(core_axis_name="c", subcore_axis_name="s")

_f32 = jnp.float32

_sc_params = pltpu.CompilerParams(use_tc_tiling_on_sc=False,
                                  needs_layout_passes=False)


def _sc_body(featp, e3, a_out, norm_out, g_hbm,
             sidx, didx, rows0, rows1, rows2, rows3, zbuf,
             onesv, degv, normv, invv, zvec, acc_sp, deg_sp,
             sem0, sem1, sem2, sem3):
    fbuf = rows2
    c = lax.axis_index("c")
    s = lax.axis_index("s")
    r0 = s * RPT
    start = s * CPT + jnp.minimum(s, XTR)
    has_extra = s < XTR

    pltpu.sync_copy(e3.at[0, pl.ds(start, CPT)], sidx.at[pl.ds(0, CPT)])
    pltpu.sync_copy(e3.at[1, pl.ds(start, CPT)], didx.at[pl.ds(0, CPT)])

    @pl.when(has_extra)
    def _():
        pltpu.sync_copy(e3.at[0, start + CPT], sidx.at[CPT])
        pltpu.sync_copy(e3.at[1, start + CPT], didx.at[CPT])

    def _fill_zb(i, _):
        for k in range(DH // 16):
            zbuf[i, pl.ds(k * 16, 16)] = jnp.zeros((16,), _f32)
        return ()

    lax.fori_loop(0, CH, _fill_zb, ())
    for k in range(CH // 16):
        onesv[pl.ds(k * 16, 16)] = jnp.ones((16,), _f32)
    for k in range(RPT // 16):
        zvec[pl.ds(k * 16, 16)] = jnp.zeros((16,), _f32)

    pltpu.sync_copy(zvec, deg_sp.at[pl.ds(r0, RPT)])
    plsc.subcore_barrier()

    def _dchunk(j, _):
        pltpu.sync_copy(onesv, deg_sp.at[didx.at[j]], add=True)
        return ()

    lax.fori_loop(0, CPT, _dchunk, ())

    @pl.when(has_extra)
    def _():
        pltpu.sync_copy(onesv, deg_sp.at[didx.at[CPT]], add=True)

    plsc.subcore_barrier()

    pltpu.sync_copy(deg_sp.at[pl.ds(r0, RPT)], degv)

    def _newton(i, _):
        sl = pl.ds(i * 16, 16)
        d = jnp.maximum(degv[sl], 1.0)
        bits = plsc.bitcast(d, jnp.int32)
        y = plsc.bitcast(jnp.int32(0x5F3759DF) - (bits >> 1), _f32)
        for _it in range(3):
            y = y * (1.5 - 0.5 * d * y * y)
        normv[sl] = y
        invv[sl] = y * y
        return ()

    lax.fori_loop(0, RPT // 16, _newton, ())

    @pl.when(c == 0)
    def _():
        pltpu.sync_copy(normv, norm_out.at[pl.ds(r0, RPT)])

    def _scale_fbuf(q, scale_ref):
        def _sgrp(i, _):
            nv = scale_ref[pl.ds(q * CH + i * 16, 16)]
            for r in range(16):
                vb = nv[jnp.full((16,), r, jnp.int32)]
                for k in range(DH // 16):
                    sl = pl.ds(k * 16, 16)
                    fbuf[i * 16 + r, sl] = fbuf[i * 16 + r, sl] * vb
            return ()

        lax.fori_loop(0, CH // 16, _sgrp, ())

    for q in range(QR):
        pltpu.sync_copy(
            featp.at[pl.ds(r0 + q * CH, CH), pl.ds(c * DH, DH)], fbuf)
        _scale_fbuf(q, normv)
        pltpu.sync_copy(fbuf, g_hbm.at[c, pl.ds(r0 + q * CH, CH)])
    for q in range(QR):
        pltpu.sync_copy(zbuf, acc_sp.at[pl.ds(r0 + q * CH, CH)])
    plsc.subcore_barrier()

    gsrc = g_hbm.at[c]
    bufs = (rows0, rows1, rows2, rows3)
    sems = (sem0, sem1, sem2, sem3)

    def _hop():
        for k in range(4):
            pltpu.async_copy(gsrc.at[sidx.at[k]], bufs[k], sems[k])

        def _quad(g, _):
            j = 4 * g
            for k in range(4):
                pltpu.make_async_copy(gsrc.at[sidx.at[j + k]], bufs[k],
                                      sems[k]).wait()
                pltpu.sync_copy(bufs[k], acc_sp.at[didx.at[j + k]],
                                add=True)

                @pl.when(j + k + 4 < CPT)
                def _():
                    pltpu.async_copy(gsrc.at[sidx.at[j + k + 4]], bufs[k],
                                     sems[k])
            return ()

        lax.fori_loop(0, CPT // 4, _quad, ())

        @pl.when(has_extra)
        def _():
            pltpu.sync_copy(gsrc.at[sidx.at[CPT]], rows0)
            pltpu.sync_copy(rows0, acc_sp.at[didx.at[CPT]], add=True)

        plsc.subcore_barrier()

    _hop()

    def _a_rows(q):
        return acc_sp.at[pl.ds(r0 + q * CH, CH)]

    for q in range(QR):
        pltpu.sync_copy(_a_rows(q), fbuf)
        _scale_fbuf(q, invv)
        pltpu.sync_copy(fbuf, g_hbm.at[c, pl.ds(r0 + q * CH, CH)])
        pltpu.sync_copy(zbuf, _a_rows(q))
    plsc.subcore_barrier()

    _hop()

    pltpu.sync_copy(acc_sp.at[pl.ds(r0, RPT)],
                    a_out.at[c, pl.ds(r0, RPT)])


_sc_call = pl.kernel(
    _sc_body,
    out_type=(
        jax.ShapeDtypeStruct((2, NP, DH), _f32),
        jax.ShapeDtypeStruct((NP,), _f32),
        jax.ShapeDtypeStruct((2, NP, DH), _f32),
    ),
    mesh=_mesh,
    scratch_types=[
        pltpu.VMEM((CPT + 1, CH), jnp.int32),
        pltpu.VMEM((CPT + 1, CH), jnp.int32),
        pltpu.VMEM((CH, DH), _f32),
        pltpu.VMEM((CH, DH), _f32),
        pltpu.VMEM((CH, DH), _f32),
        pltpu.VMEM((CH, DH), _f32),
        pltpu.VMEM((CH, DH), _f32),
        pltpu.VMEM((CH,), _f32),
        pltpu.VMEM((RPT,), _f32),
        pltpu.VMEM((RPT,), _f32),
        pltpu.VMEM((RPT,), _f32),
        pltpu.VMEM((RPT,), _f32),
        pltpu.VMEM_SHARED((NP, DH), _f32),
        pltpu.VMEM_SHARED((NP,), _f32),
        pltpu.SemaphoreType.DMA,
        pltpu.SemaphoreType.DMA,
        pltpu.SemaphoreType.DMA,
        pltpu.SemaphoreType.DMA,
    ],
    compiler_params=_sc_params,
)


R = 512


def _res_body(feat_ref, wres_ref, b_ref, out_ref):
    out_ref[...] = jnp.dot(feat_ref[...], wres_ref[...],
                           preferred_element_type=_f32) + b_ref[...]


def _tc_res(feat, W_res, bias):
    return pl.pallas_call(
        _res_body,
        grid=(NP // R,),
        in_specs=[
            pl.BlockSpec((R, D), lambda r: (r, 0)),
            pl.BlockSpec((D, D), lambda r: (0, 0)),
            pl.BlockSpec((1, D), lambda r: (0, 0)),
        ],
        out_specs=pl.BlockSpec((R, D), lambda r: (r, 0)),
        out_shape=jax.ShapeDtypeStruct((N, D), _f32),
    )(feat, W_res, bias)


def _final_body(norm_ref, a2_ref, res_ref, wfc_ref, out_ref):
    nr = norm_ref[...]
    h_lo = a2_ref[0] * nr
    h_hi = a2_ref[1] * nr
    acc = jnp.dot(h_lo, wfc_ref[pl.ds(0, DH), :],
                  preferred_element_type=_f32)
    acc += jnp.dot(h_hi, wfc_ref[pl.ds(DH, DH), :],
                   preferred_element_type=_f32)
    out_ref[...] = acc + res_ref[...]


def _tc_final(norm, a2, res, W_fc):
    return pl.pallas_call(
        _final_body,
        grid=(NP // R,),
        in_specs=[
            pl.BlockSpec((R, 1), lambda r: (r, 0)),
            pl.BlockSpec((2, R, DH), lambda r: (0, r, 0)),
            pl.BlockSpec((R, D), lambda r: (r, 0)),
            pl.BlockSpec((D, D), lambda r: (0, 0)),
        ],
        out_specs=pl.BlockSpec((R, D), lambda r: (r, 0)),
        out_shape=jax.ShapeDtypeStruct((N, D), _f32),
    )(norm, a2, res, W_fc)


def kernel(feat, edge_index, W_fc, b_fc, W_res, b_res):
    e3 = edge_index.reshape(2, NCHT, CH)
    feat_pad = jnp.pad(feat, ((0, NP - N), (0, 0)))

    bias = (b_fc + b_res)[None, :]
    res = _tc_res(feat, W_res, bias)
    a2, norm, _ = _sc_call(feat_pad, e3)
    return _tc_final(norm[:, None], a2, res, W_fc)


# speedup vs baseline: 3.0686x; 1.0288x over previous
import jax
import jax.numpy as jnp
from jax import lax
from jax.experimental import pallas as pl
from jax.experimental.pallas import tpu as pltpu
from jax.experimental.pallas import tpu_sc as plsc

N = 10000
D = 128
DH = 64
NP = 10240
NS = 16
RPT = NP // NS
CH = 128
E = 320000
NCHT = E // CH
CPT = NCHT // NS
XTR = NCHT - NS * CPT
QR = RPT // CH

_mesh = plsc.VectorSubcoreMesh(core_axis_name="c", subcore_axis_name="s")

_f32 = jnp.float32

_sc_params = pltpu.CompilerParams(use_tc_tiling_on_sc=False,
                                  needs_layout_passes=False)


def _sc_body(featp, e3, a_out, norm_out, g_hbm,
             sidx, didx, rows0, rows1, rows2, rows3, zbuf,
             onesv, degv, normv, invv, zvec, acc_sp, deg_sp,
             sem0, sem1, sem2, sem3):
    fbuf = rows2
    c = lax.axis_index("c")
    s = lax.axis_index("s")
    r0 = s * RPT
    start = s * CPT + jnp.minimum(s, XTR)
    has_extra = s < XTR

    pltpu.sync_copy(e3.at[0, pl.ds(start, CPT)], sidx.at[pl.ds(0, CPT)])
    pltpu.sync_copy(e3.at[1, pl.ds(start, CPT)], didx.at[pl.ds(0, CPT)])

    @pl.when(has_extra)
    def _():
        pltpu.sync_copy(e3.at[0, start + CPT], sidx.at[CPT])
        pltpu.sync_copy(e3.at[1, start + CPT], didx.at[CPT])

    def _fill_zb(i, _):
        for k in range(DH // 16):
            zbuf[i, pl.ds(k * 16, 16)] = jnp.zeros((16,), _f32)
        return ()

    lax.fori_loop(0, CH, _fill_zb, ())
    for k in range(CH // 16):
        onesv[pl.ds(k * 16, 16)] = jnp.ones((16,), _f32)
    for k in range(RPT // 16):
        zvec[pl.ds(k * 16, 16)] = jnp.zeros((16,), _f32)

    pltpu.sync_copy(zvec, deg_sp.at[pl.ds(r0, RPT)])
    plsc.subcore_barrier()

    dsems = (sem0, sem1, sem2, sem3)

    def _dquad(g, _):
        j = 4 * g
        for k in range(4):
            pltpu.async_copy(onesv, deg_sp.at[didx.at[j + k]], dsems[k],
                             add=True)
        for k in range(4):
            pltpu.make_async_copy(onesv, deg_sp.at[didx.at[j + k]],
                                  dsems[k]).wait()
        return ()

    lax.fori_loop(0, CPT // 4, _dquad, ())

    @pl.when(has_extra)
    def _():
        pltpu.sync_copy(onesv, deg_sp.at[didx.at[CPT]], add=True)

    plsc.subcore_barrier()

    pltpu.sync_copy(deg_sp.at[pl.ds(r0, RPT)], degv)

    def _newton(i, _):
        sl = pl.ds(i * 16, 16)
        d = jnp.maximum(degv[sl], 1.0)
        bits = plsc.bitcast(d, jnp.int32)
        y = plsc.bitcast(jnp.int32(0x5F3759DF) - (bits >> 1), _f32)
        for _it in range(3):
            y = y * (1.5 - 0.5 * d * y * y)
        normv[sl] = y
        invv[sl] = y * y
        return ()

    lax.fori_loop(0, RPT // 16, _newton, ())

    @pl.when(c == 0)
    def _():
        pltpu.sync_copy(normv, norm_out.at[pl.ds(r0, RPT)])

    def _scale_fbuf(q, scale_ref):
        def _sgrp(i, _):
            nv = scale_ref[pl.ds(q * CH + i * 16, 16)]
            for r in range(16):
                vb = nv[jnp.full((16,), r, jnp.int32)]
                for k in range(DH // 16):
                    sl = pl.ds(k * 16, 16)
                    fbuf[i * 16 + r, sl] = fbuf[i * 16 + r, sl] * vb
            return ()

        lax.fori_loop(0, CH // 16, _sgrp, ())

    for q in range(QR):
        pltpu.sync_copy(
            featp.at[pl.ds(r0 + q * CH, CH), pl.ds(c * DH, DH)], fbuf)
        _scale_fbuf(q, normv)
        pltpu.sync_copy(fbuf, g_hbm.at[c, pl.ds(r0 + q * CH, CH)])
    for q in range(QR):
        pltpu.sync_copy(zbuf, acc_sp.at[pl.ds(r0 + q * CH, CH)])
    plsc.subcore_barrier()

    gsrc = g_hbm.at[c]
    bufs = (rows0, rows1, rows2, rows3)
    sems = (sem0, sem1, sem2, sem3)

    def _hop():
        for k in range(4):
            pltpu.async_copy(gsrc.at[sidx.at[k]], bufs[k], sems[k])

        def _quad(g, _):
            j = 4 * g
            for k in range(4):
                pltpu.make_async_copy(gsrc.at[sidx.at[j + k]], bufs[k],
                                      sems[k]).wait()
                pltpu.sync_copy(bufs[k], acc_sp.at[didx.at[j + k]],
                                add=True)

                @pl.when(j + k + 4 < CPT)
                def _():
                    pltpu.async_copy(gsrc.at[sidx.at[j + k + 4]], bufs[k],
                                     sems[k])
            return ()

        lax.fori_loop(0, CPT // 4, _quad, ())

        @pl.when(has_extra)
        def _():
            pltpu.sync_copy(gsrc.at[sidx.at[CPT]], rows0)
            pltpu.sync_copy(rows0, acc_sp.at[didx.at[CPT]], add=True)

        plsc.subcore_barrier()

    _hop()

    def _a_rows(q):
        return acc_sp.at[pl.ds(r0 + q * CH, CH)]

    for q in range(QR):
        pltpu.sync_copy(_a_rows(q), fbuf)
        _scale_fbuf(q, invv)
        pltpu.sync_copy(fbuf, g_hbm.at[c, pl.ds(r0 + q * CH, CH)])
        pltpu.sync_copy(zbuf, _a_rows(q))
    plsc.subcore_barrier()

    _hop()

    pltpu.sync_copy(acc_sp.at[pl.ds(r0, RPT)],
                    a_out.at[c, pl.ds(r0, RPT)])


_sc_call = pl.kernel(
    _sc_body,
    out_type=(
        jax.ShapeDtypeStruct((2, NP, DH), _f32),
        jax.ShapeDtypeStruct((NP,), _f32),
        jax.ShapeDtypeStruct((2, NP, DH), _f32),
    ),
    mesh=_mesh,
    scratch_types=[
        pltpu.VMEM((CPT + 1, CH), jnp.int32),
        pltpu.VMEM((CPT + 1, CH), jnp.int32),
        pltpu.VMEM((CH, DH), _f32),
        pltpu.VMEM((CH, DH), _f32),
        pltpu.VMEM((CH, DH), _f32),
        pltpu.VMEM((CH, DH), _f32),
        pltpu.VMEM((CH, DH), _f32),
        pltpu.VMEM((CH,), _f32),
        pltpu.VMEM((RPT,), _f32),
        pltpu.VMEM((RPT,), _f32),
        pltpu.VMEM((RPT,), _f32),
        pltpu.VMEM((RPT,), _f32),
        pltpu.VMEM_SHARED((NP, DH), _f32),
        pltpu.VMEM_SHARED((NP,), _f32),
        pltpu.SemaphoreType.DMA,
        pltpu.SemaphoreType.DMA,
        pltpu.SemaphoreType.DMA,
        pltpu.SemaphoreType.DMA,
    ],
    compiler_params=_sc_params,
)


R = 512


def _res_body(feat_ref, wres_ref, b_ref, out_ref):
    out_ref[...] = jnp.dot(feat_ref[...], wres_ref[...],
                           preferred_element_type=_f32) + b_ref[...]


def _tc_res(feat, W_res, bias):
    return pl.pallas_call(
        _res_body,
        grid=(NP // R,),
        in_specs=[
            pl.BlockSpec((R, D), lambda r: (r, 0)),
            pl.BlockSpec((D, D), lambda r: (0, 0)),
            pl.BlockSpec((1, D), lambda r: (0, 0)),
        ],
        out_specs=pl.BlockSpec((R, D), lambda r: (r, 0)),
        out_shape=jax.ShapeDtypeStruct((N, D), _f32),
    )(feat, W_res, bias)


def _final_body(norm_ref, a2_ref, res_ref, wfc_ref, out_ref):
    nr = norm_ref[...]
    h_lo = a2_ref[0] * nr
    h_hi = a2_ref[1] * nr
    acc = jnp.dot(h_lo, wfc_ref[pl.ds(0, DH), :],
                  preferred_element_type=_f32)
    acc += jnp.dot(h_hi, wfc_ref[pl.ds(DH, DH), :],
                   preferred_element_type=_f32)
    out_ref[...] = acc + res_ref[...]


def _tc_final(norm, a2, res, W_fc):
    return pl.pallas_call(
        _final_body,
        grid=(NP // R,),
        in_specs=[
            pl.BlockSpec((R, 1), lambda r: (r, 0)),
            pl.BlockSpec((2, R, DH), lambda r: (0, r, 0)),
            pl.BlockSpec((R, D), lambda r: (r, 0)),
            pl.BlockSpec((D, D), lambda r: (0, 0)),
        ],
        out_specs=pl.BlockSpec((R, D), lambda r: (r, 0)),
        out_shape=jax.ShapeDtypeStruct((N, D), _f32),
    )(norm, a2, res, W_fc)


def kernel(feat, edge_index, W_fc, b_fc, W_res, b_res):
    e3 = edge_index.reshape(2, NCHT, CH)
    feat_pad = jnp.pad(feat, ((0, NP - N), (0, 0)))

    bias = (b_fc + b_res)[None, :]
    res = _tc_res(feat, W_res, bias)
    a2, norm, _ = _sc_call(feat_pad, e3)
    return _tc_final(norm[:, None], a2, res, W_fc)
